# Initial kernel scaffold; baseline (speedup 1.0000x reference)
import jax
import jax.numpy as jnp
from jax.experimental import pallas as pl


def kernel(data, edge_index_d4, edge_type_d4, edge_index_d5, edge_type_d5, edge_index_d6, edge_type_d6, edge_index_d7, edge_type_d7, depth, params):
    raise NotImplementedError("write your pallas kernel here")


# speedup vs baseline: 4.7554x; 4.7554x over previous
import functools

import jax
import jax.numpy as jnp
from jax import lax
from jax.experimental import pallas as pl
from jax.experimental.pallas import tpu as pltpu
from jax.experimental.pallas import tpu_sc as plsc

_NODES = {4: 256, 5: 2048, 6: 16384, 7: 131072}
_GROUPS = 8
_F32 = jnp.float32

_ACCUM_BYTES = 4 * 1024 * 1024


def _activate(x, act):
    if act == "relu":
        return jnp.maximum(x, 0.0)
    return jax.nn.gelu(x)


def _gn_rows(x, g, b):
    c = x.shape[1]
    k = c // _GROUPS
    cols = []
    for gi in range(_GROUPS):
        xs = x[:, gi * k:(gi + 1) * k]
        mu = jnp.mean(xs, axis=1, keepdims=True)
        var = jnp.mean((xs - mu) ** 2, axis=1, keepdims=True)
        cols.append((xs - mu) * lax.rsqrt(var + 1e-5))
    return jnp.concatenate(cols, axis=1) * g + b


def _rows_block(n):
    return min(n, 1024)


_DOT = dict(preferred_element_type=_F32)


@functools.lru_cache(maxsize=None)
def _gn_mm7_call(nn, c, m, npiece, act, mode):
    B = _rows_block(nn)
    grid = (nn // B,)
    mp = m // npiece

    def body(*refs):
        if mode == "x":
            x_ref, g_ref, b_ref, w_ref, o_ref = refs
            x = x_ref[...]
        else:
            p_ref, pb_ref, g_ref, b_ref, w_ref, o_ref = refs
            x = p_ref[0] + p_ref[1] + pb_ref[...]
        h = _activate(_gn_rows(x, g_ref[...], b_ref[...]), act)
        w = w_ref[...]
        for t in range(7):
            hw = jnp.dot(h, w[t], **_DOT)
            if npiece == 1:
                o_ref[t] = hw
            else:
                for kp in range(npiece):
                    o_ref[kp, t] = hw[:, kp * mp:(kp + 1) * mp]

    vec = lambda: pl.BlockSpec((1, c), lambda i: (0, 0))
    if mode == "x":
        in_specs = [pl.BlockSpec((B, c), lambda i: (i, 0)), vec(), vec(),
                    pl.BlockSpec((7, c, m), lambda i: (0, 0, 0))]
    else:
        in_specs = [pl.BlockSpec((2, B, c), lambda i: (0, i, 0)), vec(), vec(),
                    vec(), pl.BlockSpec((7, c, m), lambda i: (0, 0, 0))]
    if npiece == 1:
        out_spec = pl.BlockSpec((7, B, m), lambda i: (0, i, 0))
        out_shape = jax.ShapeDtypeStruct((7, nn, m), _F32)
    else:
        out_spec = pl.BlockSpec((npiece, 7, B, mp), lambda i: (0, 0, i, 0))
        out_shape = jax.ShapeDtypeStruct((npiece, 7, nn, mp), _F32)
    return pl.pallas_call(body, grid=grid, in_specs=in_specs,
                          out_specs=out_spec, out_shape=out_shape)


def _gn_mm7(x_or_parts, pbias, g, b, w, act, mode):
    nn = x_or_parts.shape[-2]
    c = x_or_parts.shape[-1]
    m = w.shape[2]
    dmax = max(8, _ACCUM_BYTES // (4 * nn))
    npiece = -(-m // min(m, dmax))
    f = _gn_mm7_call(nn, c, m, npiece, act, mode)
    g2, b2 = g.reshape(1, c), b.reshape(1, c)
    if mode == "x":
        out = f(x_or_parts, g2, b2, w)
    else:
        out = f(x_or_parts, pbias.reshape(1, c), g2, b2, w)
    if npiece == 1:
        return [out.reshape(7 * nn, m)]
    mp = m // npiece
    return [out[kp].reshape(7 * nn, mp) for kp in range(npiece)]


@functools.lru_cache(maxsize=None)
def _add3_call(nn, c):
    B = _rows_block(nn)

    def body(x_ref, p_ref, b_ref, o_ref):
        o_ref[...] = x_ref[...] + p_ref[0] + p_ref[1] + b_ref[...]

    return pl.pallas_call(
        body, grid=(nn // B,),
        in_specs=[pl.BlockSpec((B, c), lambda i: (i, 0)),
                  pl.BlockSpec((2, B, c), lambda i: (0, i, 0)),
                  pl.BlockSpec((1, c), lambda i: (0, 0))],
        out_specs=pl.BlockSpec((B, c), lambda i: (i, 0)),
        out_shape=jax.ShapeDtypeStruct((nn, c), _F32))


@functools.lru_cache(maxsize=None)
def _scconv_call(e, nn, d):
    NC, NS = 2, 16
    NW = NC * NS
    e_sc = e // NC
    e_tile = e // NW
    K = min(128, e_tile)
    nblk = e_tile // K
    rpt = nn // NS
    mesh = plsc.VectorSubcoreMesh(core_axis_name="c", subcore_axis_name="s")

    @functools.partial(
        pl.kernel, mesh=mesh,
        compiler_params=pltpu.CompilerParams(use_tc_tiling_on_sc=False),
        out_type=jax.ShapeDtypeStruct((NC * nn, d), _F32),
        scratch_types=[
            pltpu.VMEM((K,), jnp.int32),
            pltpu.VMEM((K,), jnp.int32),
            pltpu.VMEM((K, d), _F32),
            pltpu.VMEM_SHARED((nn, d), _F32),
            pltpu.SemaphoreType.DMA,
        ])
    def conv(table, gidx, dst, zrows, out, idxb, dstb, rows, accum, sem):
        cid = lax.axis_index("c")
        sid = lax.axis_index("s")
        r0 = sid * rpt
        pltpu.sync_copy(zrows, accum.at[pl.ds(r0, rpt)])
        plsc.subcore_barrier()

        def step(bi, carry):
            base = cid * e_sc + sid * e_tile + bi * K
            pltpu.sync_copy(gidx.at[pl.ds(base, K)], idxb)
            pltpu.sync_copy(dst.at[pl.ds(base, K)], dstb)
            pltpu.async_copy(table.at[idxb], rows, sem).wait()
            pltpu.sync_copy(rows, accum.at[dstb], add=True)
            return carry

        lax.fori_loop(0, nblk, step, 0)
        plsc.subcore_barrier()
        pltpu.sync_copy(accum.at[pl.ds(r0, rpt)],
                        out.at[pl.ds(cid * nn + r0, rpt)])

    return conv


def _scconv(tables, gidx, dst, nn):
    e = gidx.shape[0]
    outs = []
    for tb in tables:
        dp = tb.shape[1]
        f = _scconv_call(e, nn, dp)
        z = jnp.zeros((nn // 16, dp), _F32)
        o = f(tb, gidx, dst, z)
        outs.append(o.reshape(2, nn, dp))
    return outs[0] if len(outs) == 1 else jnp.concatenate(outs, axis=2)


@functools.lru_cache(maxsize=None)
def _gidx_call(e, nn):
    B = 2048
    nb = e // B

    def body(s_ref, t_ref, o_ref):
        o_ref[...] = t_ref[...] * nn + s_ref[...]

    spec = pl.BlockSpec((1, 1, B), lambda i: (i, 0, 0))
    return pl.pallas_call(
        body, grid=(nb,), in_specs=[spec, spec], out_specs=spec,
        out_shape=jax.ShapeDtypeStruct((nb, 1, B), jnp.int32))


def _gidx(src, et, nn):
    e = src.shape[0]
    nb = e // 2048
    out = _gidx_call(e, nn)(src.reshape(nb, 1, 2048), et.reshape(nb, 1, 2048))
    return out.reshape(e)


@functools.lru_cache(maxsize=None)
def _down_call(n8, c, c2):
    B = _rows_block(n8)
    vec = lambda: pl.BlockSpec((1, c2), lambda i: (0, 0))

    def body(x_ref, w_ref, b_ref, g_ref, gb_ref, o_ref):
        x3 = x_ref[...]
        xp = x3[:, 0, :]
        for i in range(1, 8):
            xp = xp + x3[:, i, :]
        y = jnp.dot(xp * 0.125, w_ref[...], **_DOT) + b_ref[...]
        o_ref[...] = _activate(_gn_rows(y, g_ref[...], gb_ref[...]), "relu")

    return pl.pallas_call(
        body, grid=(n8 // B,),
        in_specs=[pl.BlockSpec((B, 8, c), lambda i: (i, 0, 0)),
                  pl.BlockSpec((c, c2), lambda i: (0, 0)), vec(), vec(), vec()],
        out_specs=pl.BlockSpec((B, c2), lambda i: (i, 0)),
        out_shape=jax.ShapeDtypeStruct((n8, c2), _F32))


def _down(x, p):
    n, c = x.shape
    c2 = p["W"].shape[1]
    f = _down_call(n // 8, c, c2)
    return f(x.reshape(n // 8, 8, c), p["W"], p["b"].reshape(1, c2),
             p["ng"].reshape(1, c2), p["nb"].reshape(1, c2))


@functools.lru_cache(maxsize=None)
def _up_call(n, c, c2, act, with_skip):
    B = _rows_block(n)
    vec = lambda: pl.BlockSpec((1, c2), lambda i: (0, 0))

    def body(*refs):
        if with_skip:
            x_ref, w_ref, b_ref, g_ref, gb_ref, s_ref, o_ref = refs
        else:
            x_ref, w_ref, b_ref, g_ref, gb_ref, o_ref = refs
        y = jnp.dot(x_ref[...], w_ref[...], **_DOT) + b_ref[...]
        y = _activate(_gn_rows(y, g_ref[...], gb_ref[...]), act)
        y3 = jnp.broadcast_to(y[:, None, :], (B, 8, c2))
        if with_skip:
            y3 = y3 + s_ref[...]
        o_ref[...] = y3

    in_specs = [pl.BlockSpec((B, c), lambda i: (i, 0)),
                pl.BlockSpec((c, c2), lambda i: (0, 0)), vec(), vec(), vec()]
    if with_skip:
        in_specs.append(pl.BlockSpec((B, 8, c2), lambda i: (i, 0, 0)))
    return pl.pallas_call(
        body, grid=(n // B,), in_specs=in_specs,
        out_specs=pl.BlockSpec((B, 8, c2), lambda i: (i, 0, 0)),
        out_shape=jax.ShapeDtypeStruct((n, 8, c2), _F32))


def _up(x, p, act, skip):
    n, c = x.shape
    c2 = p["W"].shape[1]
    f = _up_call(n, c, c2, act, skip is not None)
    args = [x, p["W"], p["b"].reshape(1, c2), p["ng"].reshape(1, c2),
            p["nb"].reshape(1, c2)]
    if skip is not None:
        args.append(skip.reshape(n, 8, c2))
    return f(*args).reshape(8 * n, c2)


@functools.lru_cache(maxsize=None)
def _pred_call(n, c, mid, cout, act):
    B = _rows_block(n)
    vec = lambda: pl.BlockSpec((1, mid), lambda i: (0, 0))

    def body(x_ref, w1_ref, b1_ref, g_ref, gb_ref, w2_ref, b2_ref, o_ref):
        h = jnp.dot(x_ref[...], w1_ref[...], **_DOT) + b1_ref[...]
        h = _activate(_gn_rows(h, g_ref[...], gb_ref[...]), act)
        o_ref[...] = jnp.dot(h, w2_ref[...], **_DOT) + b2_ref[...]

    return pl.pallas_call(
        body, grid=(n // B,),
        in_specs=[pl.BlockSpec((B, c), lambda i: (i, 0)),
                  pl.BlockSpec((c, mid), lambda i: (0, 0)), vec(), vec(), vec(),
                  pl.BlockSpec((mid, cout), lambda i: (0, 0)),
                  pl.BlockSpec((1, cout), lambda i: (0, 0))],
        out_specs=pl.BlockSpec((B, cout), lambda i: (i, 0)),
        out_shape=jax.ShapeDtypeStruct((n, cout), _F32))


def _pred(x, p, act):
    n, c = x.shape
    mid = p["W1"].shape[1]
    cout = p["W2"].shape[1]
    f = _pred_call(n, c, mid, cout, act)
    return f(x, p["W1"], p["b1"].reshape(1, mid), p["ng"].reshape(1, mid),
             p["nb"].reshape(1, mid), p["W2"], p["b2"].reshape(1, cout))


def _gn_x(x, g, b):
    n, c = x.shape
    xg = x.reshape(n, _GROUPS, c // _GROUPS)
    mu = xg.mean(axis=2, keepdims=True)
    var = xg.var(axis=2, keepdims=True)
    xg = (xg - mu) * lax.rsqrt(var + 1e-5)
    return xg.reshape(n, c) * g + b


def _conv_x(h, w, b, gidx, dst, nn):
    hw = jnp.stack([h @ w[t] for t in range(7)]).reshape(7 * nn, w.shape[2])
    return jax.ops.segment_sum(hw[gidx], dst, num_segments=nn) + b


def _resblock_x(x, p, graph, act):
    gidx, dst, nn = graph
    h = act(_gn_x(x, p["n1g"], p["n1b"]))
    h = _conv_x(h, p["W1"], p["b1"], gidx, dst, nn)
    h = act(_gn_x(h, p["n2g"], p["n2b"]))
    h = _conv_x(h, p["W2"], p["b2"], gidx, dst, nn)
    return x + h


def _resblock(x, p, graph, act):
    gidx, dst, nn = graph
    tables1 = _gn_mm7(x, None, p["n1g"], p["n1b"], p["W1"], act, "x")
    parts1 = _scconv(tables1, gidx, dst, nn)
    tables2 = _gn_mm7(parts1, p["b1"], p["n2g"], p["n2b"], p["W2"], act,
                      "parts")
    parts2 = _scconv(tables2, gidx, dst, nn)
    c = x.shape[1]
    return _add3_call(nn, c)(x, parts2, p["b2"].reshape(1, c))


def kernel(data, edge_index_d4, edge_type_d4, edge_index_d5, edge_type_d5,
           edge_index_d6, edge_type_d6, edge_index_d7, edge_type_d7, depth,
           params):
    del depth
    relu = jax.nn.relu
    edges = {4: (edge_index_d4, edge_type_d4), 5: (edge_index_d5, edge_type_d5),
             6: (edge_index_d6, edge_type_d6), 7: (edge_index_d7, edge_type_d7)}
    graphs_x = {}
    graphs_p = {}
    for d, (ei, et) in edges.items():
        graphs_x[d] = (et * _NODES[d] + ei[0], ei[1], _NODES[d])
        if d >= 6:
            graphs_p[d] = (_gidx(ei[0], et, _NODES[d]), ei[1], _NODES[d])

    unet = params["unet"]
    out = {6: data}
    for i in range(3):
        di = 6 - i
        x = out[di]
        for rp in unet["enc"][i]:
            x = _resblock_x(x, rp, graphs_x[di], relu)
        out[di] = x
        if i < 2:
            n, c = x.shape
            xp = x.reshape(n // 8, 8, c).mean(axis=1)
            dp = unet["down"][i]
            out[di - 1] = relu(_gn_x(xp @ dp["W"] + dp["b"], dp["ng"], dp["nb"]))

    x = out[4]
    for i in range(2):
        di = 4 + i
        for rp in unet["dec"][i]:
            x = _resblock_x(x, rp, graphs_x[di], relu)
        up = unet["up"][i]
        if i == 0:
            h = relu(_gn_x(x @ up["W"] + up["b"], up["ng"], up["nb"]))
            x = jnp.repeat(h, 8, axis=0) + out[di + 1]

    x = _up(x, unet["up"][1], "relu", out[6])
    for rp in unet["dec"][2]:
        x = _resblock(x, rp, graphs_p[6], "relu")

    signals = []
    for i in range(2):
        di = 6 + i
        for rp in params["dec"]["blocks"][i]:
            x = _resblock(x, rp, graphs_p[di], "gelu")
        signals.append(_pred(x, params["dec"]["regress"][i], "gelu"))
        if i < 1:
            x = _up(x, params["dec"]["up"][0], "gelu", None)
    return tuple(signals)
